# baseline (speedup 1.0000x reference)
import functools

import jax
import jax.numpy as jnp
from jax import lax
from jax.experimental import pallas as pl
from jax.experimental.pallas import tpu as pltpu
from jax.experimental.pallas import tpu_sc as plsc

_INFO = plsc.get_sparse_core_info()
_NC = _INFO.num_cores
_NS = _INFO.num_subcores
_NW = _NC * _NS
_IDX_W = 128


def _make_gather(num_nodes: int, dim: int, batch: int):
    assert batch % (_NW * _IDX_W) == 0
    ch = batch // (_NW * _IDX_W)
    nrows = _NW * ch

    mesh = plsc.VectorSubcoreMesh(core_axis_name="c", subcore_axis_name="s")

    @functools.partial(
        pl.kernel,
        mesh=mesh,
        out_type=(
            jax.ShapeDtypeStruct((nrows, _IDX_W, dim), jnp.float32),
            jax.ShapeDtypeStruct((nrows, _IDX_W), jnp.int32),
        ),
        scratch_types=[
            pltpu.VMEM((ch, _IDX_W), jnp.int32),
            pltpu.VMEM((_IDX_W, dim), jnp.float32),
            pltpu.VMEM((_IDX_W,), jnp.int32),
            pltpu.SemaphoreType.DMA,
            pltpu.SemaphoreType.DMA,
        ],
    )
    def k(mem_hbm, idx_hbm, lu_hbm, mem_out, lu_out,
          idx_v, rows_v, lu_v, sem_m, sem_l):
        wid = lax.axis_index("s") * _NC + lax.axis_index("c")
        base = wid * ch
        pltpu.sync_copy(idx_hbm.at[pl.ds(base, ch)], idx_v)

        def body(j, carry):
            m = pltpu.async_copy(mem_hbm.at[idx_v.at[j]], rows_v, sem_m)
            l = pltpu.async_copy(lu_hbm.at[idx_v.at[j]], lu_v, sem_l)
            m.wait()
            l.wait()
            pltpu.sync_copy(rows_v, mem_out.at[base + j])
            pltpu.sync_copy(lu_v, lu_out.at[base + j])
            return carry

        lax.fori_loop(0, ch, body, 0)

    return k


def kernel(n_id, memory, last_update):
    batch = n_id.shape[0]
    num_nodes, dim = memory.shape
    idx2d = n_id.reshape(batch // _IDX_W, _IDX_W)
    mem3, lu2 = _make_gather(num_nodes, dim, batch)(memory, idx2d, last_update)
    return mem3.reshape(batch, dim), lu2.reshape(batch)


# speedup vs baseline: 1.0940x; 1.0940x over previous
import functools

import jax
import jax.numpy as jnp
from jax import lax
from jax.experimental import pallas as pl
from jax.experimental.pallas import tpu as pltpu
from jax.experimental.pallas import tpu_sc as plsc

_INFO = plsc.get_sparse_core_info()
_NC = _INFO.num_cores
_NS = _INFO.num_subcores
_NW = _NC * _NS
_IDX_W = 128


def _make_gather(num_nodes: int, dim: int, batch: int):
    assert batch % (_NW * _IDX_W) == 0
    ch = batch // (_NW * _IDX_W)
    nrows = _NW * ch

    mesh = plsc.VectorSubcoreMesh(core_axis_name="c", subcore_axis_name="s")

    @functools.partial(
        pl.kernel,
        mesh=mesh,
        out_type=(
            jax.ShapeDtypeStruct((nrows, _IDX_W, dim), jnp.float32),
            jax.ShapeDtypeStruct((nrows, _IDX_W), jnp.int32),
        ),
        scratch_types=[
            pltpu.VMEM((ch, _IDX_W), jnp.int32),
            pltpu.VMEM((ch, _IDX_W, dim), jnp.float32),
            pltpu.VMEM((ch, _IDX_W), jnp.int32),
            pltpu.SemaphoreType.DMA,
            pltpu.SemaphoreType.DMA,
            pltpu.SemaphoreType.DMA,
        ],
    )
    def k(mem_hbm, idx_hbm, lu_hbm, mem_out, lu_out,
          idx_v, rows_v, lu_v, sem_m, sem_l, sem_w):
        wid = lax.axis_index("s") * _NC + lax.axis_index("c")
        base = wid * ch
        pltpu.sync_copy(idx_hbm.at[pl.ds(base, ch)], idx_v)
        mcps = []
        lcps = []
        for j in range(ch):
            mcps.append(pltpu.async_copy(mem_hbm.at[idx_v.at[j]],
                                         rows_v.at[j], sem_m))
            lcps.append(pltpu.async_copy(lu_hbm.at[idx_v.at[j]],
                                         lu_v.at[j], sem_l))
        wcps = []
        for j in range(ch):
            mcps[j].wait()
            wcps.append(pltpu.async_copy(rows_v.at[j],
                                         mem_out.at[base + j], sem_w))
        for j in range(ch):
            lcps[j].wait()
        wl = pltpu.async_copy(lu_v, lu_out.at[pl.ds(base, ch)], sem_l)
        for w in wcps:
            w.wait()
        wl.wait()

    return k


def kernel(n_id, memory, last_update):
    batch = n_id.shape[0]
    num_nodes, dim = memory.shape
    idx2d = n_id.reshape(batch // _IDX_W, _IDX_W)
    mem3, lu2 = _make_gather(num_nodes, dim, batch)(memory, idx2d, last_update)
    return mem3.reshape(batch, dim), lu2.reshape(batch)
